# baseline (speedup 1.0000x reference)
import functools

import jax
import jax.numpy as jnp
from jax import lax
from jax.experimental import pallas as pl
from jax.experimental.pallas import tpu as pltpu
from jax.experimental.pallas import tpu_sc as plsc

D = 64
SCALE = 8.0
NC, NS = 2, 16
NW = NC * NS
C = 128
G = 4
CG = C * G
NBUF = 3


TL = 512
NBLK = 977
H = NBLK * TL


def _detranspose_table(table_t):
    def body(a_ref, b_ref, o_ref):
        o_ref[:, 0:64] = a_ref[...].T
        o_ref[:, 64:128] = b_ref[...].T

    return pl.pallas_call(
        body,
        grid=(NBLK,),
        in_specs=[
            pl.BlockSpec((64, TL), lambda i: (0, i)),
            pl.BlockSpec((64, TL), lambda i: (0, i + NBLK)),
        ],
        out_specs=pl.BlockSpec((TL, 128), lambda i: (i, 0)),
        out_shape=jax.ShapeDtypeStruct((H, 128), jnp.float32),
    )(table_t, table_t)


@functools.partial(jax.jit, static_argnames=("nj",))
def _sc_gather_scale(x_w, table, nj):
    n = NW * nj * CG

    @functools.partial(
        pl.kernel,
        out_type=jax.ShapeDtypeStruct((n, D), jnp.float32),
        mesh=plsc.VectorSubcoreMesh(core_axis_name="c", subcore_axis_name="s"),
        scratch_types=[
            pltpu.VMEM((nj * G, C), jnp.int32),
            pltpu.VMEM((CG, D), jnp.float32),
            pltpu.VMEM((CG, D), jnp.float32),
            pltpu.VMEM((CG, D), jnp.float32),
            pltpu.SemaphoreType.DMA,
            pltpu.SemaphoreType.DMA,
            pltpu.SemaphoreType.DMA,
        ],
        compiler_params=pltpu.CompilerParams(use_tc_tiling_on_sc=False),
    )
    def body(x_hbm, tab_hbm, out_hbm, idx_v, r0, r1, r2, s0, s1, s2):
        wid = lax.axis_index("s") * NC + lax.axis_index("c")
        base = wid * (nj * CG)
        bufs = (r0, r1, r2)
        sems = (s0, s1, s2)

        pltpu.sync_copy(x_hbm.at[wid], idx_v)

        def fire(j, b):
            for g in range(G):
                pltpu.async_copy(
                    tab_hbm.at[idx_v.at[j * G + g]],
                    bufs[b].at[pl.ds(g * C, C)],
                    sems[b],
                )

        def drain_gather(j, b):
            for g in range(G):
                pltpu.make_async_copy(
                    tab_hbm.at[idx_v.at[j * G + g]],
                    bufs[b].at[pl.ds(g * C, C)],
                    sems[b],
                ).wait()

        def scale(b):
            @plsc.parallel_loop(0, CG, 1, unroll=8)
            def _(r):
                for k in range(D // 16):
                    sl = pl.ds(k * 16, 16)
                    bufs[b][r, sl] = bufs[b][r, sl] * SCALE

        def store(j, b):
            pltpu.async_copy(
                bufs[b], out_hbm.at[pl.ds(base + j * CG, CG)], sems[b]
            )

        def drain_store(j, b):
            pltpu.make_async_copy(
                bufs[b], out_hbm.at[pl.ds(base + j * CG, CG)], sems[b]
            ).wait()

        def block(jj, bmod, fire_next=True, drain_prev=True):
            nb = (bmod + 1) % NBUF
            if drain_prev:
                drain_store(jj - 2, nb)
            if fire_next:
                fire(jj + 1, nb)
            drain_gather(jj, bmod)
            scale(bmod)
            store(jj, bmod)

        fire(0, 0)
        block(0, 0, drain_prev=False)
        block(1, 1, drain_prev=False)

        def triple(s, carry):
            j = 2 + 3 * s
            block(j, 2)
            block(j + 1, 0)
            block(j + 2, 1)
            return carry

        lax.fori_loop(0, (nj - 5) // 3, triple, None)

        block(nj - 3, (nj - 3) % NBUF)
        block(nj - 2, (nj - 2) % NBUF)
        block(nj - 1, (nj - 1) % NBUF, fire_next=False)
        drain_store(nj - 2, (nj - 2) % NBUF)
        drain_store(nj - 1, (nj - 1) % NBUF)

    return body(x_w, table)


def kernel(x, input_embedding):
    b, h = x.shape
    n = b * h
    nj = n // (NW * CG)
    xi = x.astype(jnp.int32)
    x2 = jnp.where(xi < H, xi * 2, (xi - H) * 2 + 1)
    x_w = x2.reshape(NW, nj * G, C)
    tab2 = _detranspose_table(input_embedding.T)
    tab_rm = tab2.reshape(2 * H, D)
    out = _sc_gather_scale(x_w, tab_rm, nj)
    return out.reshape(b, h, D)


# speedup vs baseline: 1.2700x; 1.2700x over previous
import functools

import jax
import jax.numpy as jnp
from jax import lax
from jax.experimental import pallas as pl
from jax.experimental.pallas import tpu as pltpu
from jax.experimental.pallas import tpu_sc as plsc

D = 64
SCALE = 8.0
NC, NS = 2, 16
NW = NC * NS
C = 128
G = 4
CG = C * G
NBUF = 3


TL = 1024
NBLK = 489
H = NBLK * TL
NLAST = 976


def _detranspose_table(table_t):
    def body(a_ref, b_ref, o_ref):
        eye = jnp.eye(128, dtype=jnp.float32)
        dn = (((0,), (0,)), ((), ()))
        c = jnp.concatenate([a_ref[...], b_ref[...]], axis=0)
        o_ref[...] = jax.lax.dot_general(
            c, eye, dn, preferred_element_type=jnp.float32
        )

    return pl.pallas_call(
        body,
        grid=(NBLK,),
        in_specs=[
            pl.BlockSpec((64, TL), lambda i: (0, i)),
            pl.BlockSpec((64, TL), lambda i: (0, jnp.minimum(i + NBLK, NLAST))),
        ],
        out_specs=pl.BlockSpec((TL, 128), lambda i: (i, 0)),
        out_shape=jax.ShapeDtypeStruct((H, 128), jnp.float32),
    )(table_t, table_t)


@functools.partial(jax.jit, static_argnames=("nj",))
def _sc_gather_scale(x_w, table, nj):
    n = NW * nj * CG

    @functools.partial(
        pl.kernel,
        out_type=jax.ShapeDtypeStruct((n, D), jnp.float32),
        mesh=plsc.VectorSubcoreMesh(core_axis_name="c", subcore_axis_name="s"),
        scratch_types=[
            pltpu.VMEM((nj * G, C), jnp.int32),
            pltpu.VMEM((CG, D), jnp.float32),
            pltpu.VMEM((CG, D), jnp.float32),
            pltpu.VMEM((CG, D), jnp.float32),
            pltpu.SemaphoreType.DMA,
            pltpu.SemaphoreType.DMA,
            pltpu.SemaphoreType.DMA,
        ],
        compiler_params=pltpu.CompilerParams(use_tc_tiling_on_sc=False),
    )
    def body(x_hbm, tab_hbm, out_hbm, idx_v, r0, r1, r2, s0, s1, s2):
        wid = lax.axis_index("s") * NC + lax.axis_index("c")
        base = wid * (nj * CG)
        bufs = (r0, r1, r2)
        sems = (s0, s1, s2)

        pltpu.sync_copy(x_hbm.at[wid], idx_v)

        def fire(j, b):
            for g in range(G):
                pltpu.async_copy(
                    tab_hbm.at[idx_v.at[j * G + g]],
                    bufs[b].at[pl.ds(g * C, C)],
                    sems[b],
                )

        def drain_gather(j, b):
            for g in range(G):
                pltpu.make_async_copy(
                    tab_hbm.at[idx_v.at[j * G + g]],
                    bufs[b].at[pl.ds(g * C, C)],
                    sems[b],
                ).wait()

        def scale(b):
            @plsc.parallel_loop(0, CG, 1, unroll=8)
            def _(r):
                for k in range(D // 16):
                    sl = pl.ds(k * 16, 16)
                    bufs[b][r, sl] = bufs[b][r, sl] * SCALE

        def store(j, b):
            pltpu.async_copy(
                bufs[b], out_hbm.at[pl.ds(base + j * CG, CG)], sems[b]
            )

        def drain_store(j, b):
            pltpu.make_async_copy(
                bufs[b], out_hbm.at[pl.ds(base + j * CG, CG)], sems[b]
            ).wait()

        def block(jj, bmod, fire_next=True, drain_prev=True):
            nb = (bmod + 1) % NBUF
            if drain_prev:
                drain_store(jj - 2, nb)
            if fire_next:
                fire(jj + 1, nb)
            drain_gather(jj, bmod)
            scale(bmod)
            store(jj, bmod)

        fire(0, 0)
        block(0, 0, drain_prev=False)
        block(1, 1, drain_prev=False)

        def triple(s, carry):
            j = 2 + 3 * s
            block(j, 2)
            block(j + 1, 0)
            block(j + 2, 1)
            return carry

        lax.fori_loop(0, (nj - 5) // 3, triple, None)

        block(nj - 3, (nj - 3) % NBUF)
        block(nj - 2, (nj - 2) % NBUF)
        block(nj - 1, (nj - 1) % NBUF, fire_next=False)
        drain_store(nj - 2, (nj - 2) % NBUF)
        drain_store(nj - 1, (nj - 1) % NBUF)

    return body(x_w, table)


def kernel(x, input_embedding):
    b, h = x.shape
    n = b * h
    nj = n // (NW * CG)
    xi = x.astype(jnp.int32)
    x2 = jnp.where(xi < H, xi * 2, (xi - H) * 2 + 1)
    x_w = x2.reshape(NW, nj * G, C)
    tab2 = _detranspose_table(input_embedding.T)
    tab_rm = tab2.reshape(2 * H, D)
    out = _sc_gather_scale(x_w, tab_rm, nj)
    return out.reshape(b, h, D)
